# baseline (speedup 1.0000x reference)
import functools
import math

import jax
import jax.numpy as jnp
from jax import lax
from jax.experimental import pallas as pl
from jax.experimental.pallas import tpu as pltpu
from jax.experimental.pallas import tpu_sc as plsc

KTOP = 4

_NC = 2
_NS = 16
_NW = _NC * _NS


def _proj_body(x_ref, w_ref, b_ref, q_ref, k_ref, *, kq):
    xb = x_ref[0]
    qk = jnp.dot(xb, w_ref[...], preferred_element_type=jnp.float32)
    qk = qk + b_ref[...]
    q_ref[...] = qk[:, :kq]
    k_ref[...] = qk[:, kq:]


def _proj_call(x, b, B, T, D, kq, bta, wqk, bqk):
    return pl.pallas_call(
        functools.partial(_proj_body, kq=kq),
        grid=(T // bta,),
        in_specs=[
            pl.BlockSpec((1, bta, D), lambda i, _b=b: (_b, i, 0)),
            pl.BlockSpec((D, 2 * kq), lambda i: (0, 0)),
            pl.BlockSpec((1, 2 * kq), lambda i: (0, 0)),
        ],
        out_specs=[
            pl.BlockSpec((bta, kq), lambda i: (i, 0)),
            pl.BlockSpec((bta, kq), lambda i: (i, 0)),
        ],
        out_shape=[
            jax.ShapeDtypeStruct((T, kq), jnp.float32),
            jax.ShapeDtypeStruct((T, kq), jnp.float32),
        ],
    )(x, wqk, bqk)


def _topk_body(q_ref, k_ref, ti_ref, sv_ref, gi_ref, *, t, kq, row_off):
    q = q_ref[...]
    k = k_ref[...]
    s = lax.dot_general(q, k, (((1,), (1,)), ((), ())),
                        preferred_element_type=jnp.float32)
    s = s * jnp.float32(1.0 / math.sqrt(kq))
    col = lax.broadcasted_iota(jnp.int32, s.shape, 1)
    colf = col.astype(jnp.float32)
    neg = jnp.float32(-jnp.inf)
    big = jnp.float32(t)
    idxs, vals = [], []
    for _ in range(KTOP):
        m = jnp.max(s, axis=1, keepdims=True)
        hit = s == m
        idxf = jnp.min(jnp.where(hit, colf, big), axis=1, keepdims=True)
        idx = idxf.astype(jnp.int32)
        idxs.append(idx)
        vals.append(m)
        s = jnp.where(col == idx, neg, s)
    ti = jnp.concatenate(idxs, axis=1)
    ti_ref[...] = ti
    sv_ref[...] = jnp.concatenate(vals, axis=1)
    gi_ref[...] = ti + row_off


def _topk_call(q, k, b, half, t, kq, bt):
    t2 = t // 2
    blk0 = half * (t2 // bt)
    return pl.pallas_call(
        functools.partial(_topk_body, t=t, kq=kq, row_off=b * t),
        grid=(t2 // bt,),
        in_specs=[
            pl.BlockSpec((bt, kq), lambda i, _o=blk0: (_o + i, 0)),
            pl.BlockSpec((t, kq), lambda i: (0, 0)),
        ],
        out_specs=[
            pl.BlockSpec((bt, KTOP), lambda i: (i, 0)),
            pl.BlockSpec((bt, KTOP), lambda i: (i, 0)),
            pl.BlockSpec((bt, KTOP), lambda i: (i, 0)),
        ],
        out_shape=[
            jax.ShapeDtypeStruct((t2, KTOP), jnp.int32),
            jax.ShapeDtypeStruct((t2, KTOP), jnp.float32),
            jax.ShapeDtypeStruct((t2, KTOP), jnp.int32),
        ],
    )(q, k)


def _make_gather(b, t_base, t_span, D, chunk_t):
    per_w = t_span // _NW
    n_pair = per_w // (2 * chunk_t)
    rows = chunk_t * KTOP
    mesh = plsc.VectorSubcoreMesh(
        core_axis_name="c", subcore_axis_name="s",
        num_cores=_NC, num_subcores=_NS)

    @functools.partial(
        pl.kernel, mesh=mesh,
        out_type=(),
        scratch_types=[
            pltpu.VMEM((rows,), jnp.int32),
            pltpu.VMEM((rows,), jnp.int32),
            pltpu.VMEM((rows, D), jnp.float32),
            pltpu.VMEM((rows, D), jnp.float32),
            pltpu.SemaphoreType.DMA,
            pltpu.SemaphoreType.DMA,
            pltpu.SemaphoreType.DMA,
            pltpu.SemaphoreType.DMA,
        ],
        compiler_params=pltpu.CompilerParams(use_tc_tiling_on_sc=True),
    )
    def _gather(tab_hbm, idx_hbm, out_ref,
                idx_a, idx_b, rows_a, rows_b, gsem_a, gsem_b, wsem_a, wsem_b):
        wid = lax.axis_index("s") * _NC + lax.axis_index("c")
        base = wid * per_w

        def load_and_gather(t0, idx_v, rows_v, gsem):
            pltpu.sync_copy(idx_hbm.at[pl.ds(t0 * KTOP, rows)], idx_v)
            return pltpu.async_copy(tab_hbm.at[idx_v], rows_v, gsem)

        def write_chunk(t0, rows_v, wsem):
            cps = [pltpu.async_copy(rows_v.at[pl.ds(j * KTOP, KTOP)],
                                    out_ref.at[b, t_base + t0 + j], wsem)
                   for j in range(chunk_t)]
            return cps

        load_and_gather(base, idx_a, rows_a, gsem_a)

        def body(p, carry):
            t0 = base + 2 * p * chunk_t
            load_and_gather(t0 + chunk_t, idx_b, rows_b, gsem_b)
            pltpu.make_async_copy(tab_hbm.at[idx_a], rows_a, gsem_a).wait()
            cps_a = write_chunk(t0, rows_a, wsem_a)
            for c in cps_a:
                c.wait()
            @pl.when(p < n_pair - 1)
            def _():
                load_and_gather(t0 + 2 * chunk_t, idx_a, rows_a, gsem_a)

            pltpu.make_async_copy(tab_hbm.at[idx_b], rows_b, gsem_b).wait()
            cps_b = write_chunk(t0 + chunk_t, rows_b, wsem_b)
            for c in cps_b:
                c.wait()
            return carry

        lax.fori_loop(0, n_pair, body, 0)

    return _gather


def kernel(x, Wq, bq, Wk, bk):
    B, T, D = x.shape
    KQ = Wq.shape[1]
    BTA = 512
    BT = 256

    wqk = jnp.concatenate([Wq, Wk], axis=1)
    bqk = jnp.concatenate([bq, bk]).reshape(1, 2 * KQ)

    tab = x.reshape(B * T, D)
    out_ref = jax.new_ref(lax.empty((B, T, KTOP, D), jnp.float32))

    T2 = T // 2
    tis, svs = [], []
    for b in range(B):
        q_b, k_b = _proj_call(x, b, B, T, D, KQ, BTA, wqk, bqk)
        ti_h, sv_h = [], []
        for h in range(2):
            ti_q, sv_q, gi_q = _topk_call(q_b, k_b, b, h, T, KQ, BT)
            _make_gather(b, h * T2, T2, D, 8)(
                tab, gi_q.reshape(T2 * KTOP), out_ref)
            ti_h.append(ti_q)
            sv_h.append(sv_q)
        tis.append(jnp.concatenate(ti_h, axis=0))
        svs.append(jnp.concatenate(sv_h, axis=0))

    gathered = out_ref[...]
    ti = jnp.stack(tis, axis=0)
    sv = jnp.stack(svs, axis=0)
    return gathered, ti, sv


# speedup vs baseline: 1.0550x; 1.0550x over previous
import functools
import math

import jax
import jax.numpy as jnp
from jax import lax
from jax.experimental import pallas as pl
from jax.experimental.pallas import tpu as pltpu
from jax.experimental.pallas import tpu_sc as plsc

KTOP = 4

_NC = 2
_NS = 16
_NW = _NC * _NS


def _proj_body(x_ref, w_ref, b_ref, q_ref, k_ref, *, kq):
    xb = x_ref[0]
    qk = jnp.dot(xb, w_ref[...], preferred_element_type=jnp.float32)
    qk = qk + b_ref[...]
    q_ref[0] = qk[:, :kq]
    k_ref[0] = qk[:, kq:]


def _topk_body(q_ref, k_ref, ti_ref, sv_ref, gi_ref, *, t, kq, row_off):
    q = q_ref[0]
    k = k_ref[0]
    s = lax.dot_general(q, k, (((1,), (1,)), ((), ())),
                        preferred_element_type=jnp.float32)
    s = s * jnp.float32(1.0 / math.sqrt(kq))
    col = lax.broadcasted_iota(jnp.int32, s.shape, 1)
    colf = col.astype(jnp.float32)
    neg = jnp.float32(-jnp.inf)
    big = jnp.float32(t)
    idxs, vals = [], []
    for _ in range(KTOP):
        m = jnp.max(s, axis=1, keepdims=True)
        hit = s == m
        idxf = jnp.min(jnp.where(hit, colf, big), axis=1, keepdims=True)
        idx = idxf.astype(jnp.int32)
        idxs.append(idx)
        vals.append(m)
        s = jnp.where(col == idx, neg, s)
    ti = jnp.concatenate(idxs, axis=1)
    ti_ref[...] = ti
    sv_ref[...] = jnp.concatenate(vals, axis=1)
    gi_ref[...] = ti + row_off


def _topk_call(q, k, b, half, t, kq, bt):
    t2 = t // 2
    blk0 = half * (t2 // bt)
    return pl.pallas_call(
        functools.partial(_topk_body, t=t, kq=kq, row_off=b * t),
        grid=(t2 // bt,),
        in_specs=[
            pl.BlockSpec((1, bt, kq), lambda i, _b=b, _o=blk0: (_b, _o + i, 0)),
            pl.BlockSpec((1, t, kq), lambda i, _b=b: (_b, 0, 0)),
        ],
        out_specs=[
            pl.BlockSpec((bt, KTOP), lambda i: (i, 0)),
            pl.BlockSpec((bt, KTOP), lambda i: (i, 0)),
            pl.BlockSpec((bt, KTOP), lambda i: (i, 0)),
        ],
        out_shape=[
            jax.ShapeDtypeStruct((t2, KTOP), jnp.int32),
            jax.ShapeDtypeStruct((t2, KTOP), jnp.float32),
            jax.ShapeDtypeStruct((t2, KTOP), jnp.int32),
        ],
    )(q, k)


def _make_gather(b, t_base, t_span, D, chunk_t):
    per_w = t_span // _NW
    n_pair = per_w // (2 * chunk_t)
    rows = chunk_t * KTOP
    mesh = plsc.VectorSubcoreMesh(
        core_axis_name="c", subcore_axis_name="s",
        num_cores=_NC, num_subcores=_NS)

    @functools.partial(
        pl.kernel, mesh=mesh,
        out_type=(),
        scratch_types=[
            pltpu.VMEM((rows,), jnp.int32),
            pltpu.VMEM((rows,), jnp.int32),
            pltpu.VMEM((rows, D), jnp.float32),
            pltpu.VMEM((rows, D), jnp.float32),
            pltpu.SemaphoreType.DMA,
            pltpu.SemaphoreType.DMA,
            pltpu.SemaphoreType.DMA,
            pltpu.SemaphoreType.DMA,
        ],
        compiler_params=pltpu.CompilerParams(use_tc_tiling_on_sc=True),
    )
    def _gather(tab_hbm, idx_hbm, out_ref,
                idx_a, idx_b, rows_a, rows_b, gsem_a, gsem_b, wsem_a, wsem_b):
        wid = lax.axis_index("s") * _NC + lax.axis_index("c")
        base = wid * per_w

        def load_and_gather(t0, idx_v, rows_v, gsem):
            pltpu.sync_copy(idx_hbm.at[pl.ds(t0 * KTOP, rows)], idx_v)
            return pltpu.async_copy(tab_hbm.at[idx_v], rows_v, gsem)

        def write_chunk(t0, rows_v, wsem):
            cps = [pltpu.async_copy(rows_v.at[pl.ds(j * KTOP, KTOP)],
                                    out_ref.at[b, t_base + t0 + j], wsem)
                   for j in range(chunk_t)]
            return cps

        load_and_gather(base, idx_a, rows_a, gsem_a)

        def body(p, carry):
            t0 = base + 2 * p * chunk_t
            load_and_gather(t0 + chunk_t, idx_b, rows_b, gsem_b)
            pltpu.make_async_copy(tab_hbm.at[idx_a], rows_a, gsem_a).wait()
            cps_a = write_chunk(t0, rows_a, wsem_a)
            for c in cps_a:
                c.wait()
            @pl.when(p < n_pair - 1)
            def _():
                load_and_gather(t0 + 2 * chunk_t, idx_a, rows_a, gsem_a)

            pltpu.make_async_copy(tab_hbm.at[idx_b], rows_b, gsem_b).wait()
            cps_b = write_chunk(t0 + chunk_t, rows_b, wsem_b)
            for c in cps_b:
                c.wait()
            return carry

        lax.fori_loop(0, n_pair, body, 0)

    return _gather


def kernel(x, Wq, bq, Wk, bk):
    B, T, D = x.shape
    KQ = Wq.shape[1]
    BTA = 512
    BT = 256

    wqk = jnp.concatenate([Wq, Wk], axis=1)
    bqk = jnp.concatenate([bq, bk]).reshape(1, 2 * KQ)

    q, k = pl.pallas_call(
        functools.partial(_proj_body, kq=KQ),
        grid=(B, T // BTA),
        in_specs=[
            pl.BlockSpec((1, BTA, D), lambda b, i: (b, i, 0)),
            pl.BlockSpec((D, 2 * KQ), lambda b, i: (0, 0)),
            pl.BlockSpec((1, 2 * KQ), lambda b, i: (0, 0)),
        ],
        out_specs=[
            pl.BlockSpec((1, BTA, KQ), lambda b, i: (b, i, 0)),
            pl.BlockSpec((1, BTA, KQ), lambda b, i: (b, i, 0)),
        ],
        out_shape=[
            jax.ShapeDtypeStruct((B, T, KQ), jnp.float32),
            jax.ShapeDtypeStruct((B, T, KQ), jnp.float32),
        ],
    )(x, wqk, bqk)

    tab = x.reshape(B * T, D)
    out_ref = jax.new_ref(lax.empty((B, T, KTOP, D), jnp.float32))

    T2 = T // 2
    tis, svs = [], []
    for b in range(B):
        ti_h, sv_h = [], []
        for h in range(2):
            ti_q, sv_q, gi_q = _topk_call(q, k, b, h, T, KQ, BT)
            _make_gather(b, h * T2, T2, D, 8)(
                tab, gi_q.reshape(T2 * KTOP), out_ref)
            ti_h.append(ti_q)
            sv_h.append(sv_q)
        tis.append(jnp.concatenate(ti_h, axis=0))
        svs.append(jnp.concatenate(sv_h, axis=0))

    gathered = out_ref[...]
    ti = jnp.stack(tis, axis=0)
    sv = jnp.stack(svs, axis=0)
    return gathered, ti, sv
